# baseline (speedup 1.0000x reference)
import functools

import jax
import jax.numpy as jnp
from jax import lax
from jax.experimental import pallas as pl
from jax.experimental.pallas import tpu as pltpu
from jax.experimental.pallas import tpu_sc as plsc

VOCAB = 100000
EMBED = 64
N_TOPICS = 25
NUM_SAMPLED = 15
WINDOW = 20
BATCH = 4096
LAMBDA_CONST = 100.0
EPSILON = 1e-09

NC, NS, LANES = 2, 16, 16
NW = NC * NS
BPW = BATCH // NW
RPB = WINDOW * (1 + NUM_SAMPLED)
RPAD = 384
NCH = RPAD // 128
GROUPS = RPB // LANES
TPAD = 32


_GDN = lax.GatherDimensionNumbers(
    offset_dims=(), collapsed_slice_dims=(0,), start_index_map=(0,))


def _permute(v, idx):
    return lax.gather(v, idx[:, None], _GDN, (1,),
                      mode=lax.GatherScatterMode.PROMISE_IN_BOUNDS)


def _bcast(v, lane):
    return _permute(v, jnp.full((LANES,), lane, dtype=jnp.int32))


def _vsum16(v, lane):
    for sh in (8, 4, 2, 1):
        v = v + _permute(v, lane ^ sh)
    return v


def _vmax16(v, lane):
    for sh in (8, 4, 2, 1):
        v = jnp.maximum(v, _permute(v, lane ^ sh))
    return v


def _sc_body(r_hbm, u_hbm, didx_hbm, pidx_hbm, wv_hbm, dwt_hbm, tv_hbm,
             lw_hbm, aq_hbm, aj_hbm,
             dots_hbm, dwo_hbm, wo_hbm,
             didx_v, pidx_v, dw_v, w_v, tv_v, ctx_v,
             r_v, u_v, q_v, j_v, cidx_v,
             rows0_v, rows1_v, rows2_v, rows3_v, dots4_v,
             sem, sem_qj, sem_rows):
    wid = lax.axis_index("s") * NC + lax.axis_index("c")
    b0 = wid * BPW
    lane = lax.broadcasted_iota(jnp.int32, (LANES,), 0)

    pltpu.sync_copy(didx_hbm.at[pl.ds(b0, BPW)], didx_v)
    pltpu.sync_copy(pidx_hbm.at[pl.ds(b0, BPW)], pidx_v)
    pltpu.sync_copy(tv_hbm, tv_v)
    cp1 = pltpu.async_copy(dwt_hbm.at[didx_v], dw_v, sem)
    cp2 = pltpu.async_copy(lw_hbm.at[didx_v], w_v, sem)
    cp3 = pltpu.async_copy(wv_hbm.at[pidx_v], ctx_v, sem)
    cp1.wait()
    cp2.wait()
    cp3.wait()
    pltpu.sync_copy(dw_v, dwo_hbm.at[pl.ds(b0, BPW)])
    pltpu.sync_copy(w_v, wo_hbm.at[pl.ds(b0, BPW)])

    def ctx_body(i, carry):
        c0 = dw_v[i, pl.ds(0, LANES)]
        c1 = dw_v[i, pl.ds(LANES, LANES)]
        valid1 = lane < (N_TOPICS - LANES)
        neg_big = jnp.float32(-1e30)
        m = _vmax16(jnp.maximum(c0, jnp.where(valid1, c1, neg_big)), lane)
        e0 = jnp.exp(c0 - m)
        e1 = jnp.where(valid1, jnp.exp(c1 - m), jnp.float32(0.0))
        inv = 1.0 / _vsum16(e0 + e1, lane)
        mix = [jnp.zeros((LANES,), jnp.float32) for _ in range(EMBED // LANES)]
        for t in range(N_TOPICS):
            src = e0 if t < LANES else e1
            ln = t if t < LANES else t - LANES
            p = _bcast(src, ln) * inv
            for j in range(EMBED // LANES):
                mix[j] = mix[j] + p * tv_v[t, pl.ds(LANES * j, LANES)]
        for j in range(EMBED // LANES):
            sl = pl.ds(LANES * j, LANES)
            ctx_v[i, sl] = mix[j] + ctx_v[i, sl]
        return carry

    lax.fori_loop(0, BPW, ctx_body, 0)

    rows_bufs = (rows0_v, rows1_v, rows2_v, rows3_v)

    def s1(k, par):
        kk = jnp.minimum(k, BPW - 1)
        pltpu.sync_copy(r_hbm.at[b0 + kk], r_v.at[par])
        pltpu.sync_copy(u_hbm.at[b0 + kk], u_v.at[par])
        for c in range(NCH):
            pltpu.async_copy(aq_hbm.at[r_v.at[par, c]], q_v.at[par, c], sem_qj)
            pltpu.async_copy(aj_hbm.at[r_v.at[par, c]], j_v.at[par, c], sem_qj)

    def s1_drain(par):
        for c in range(NCH):
            pltpu.make_async_copy(aq_hbm.at[r_v.at[par, c]],
                                  q_v.at[par, c], sem_qj).wait()
            pltpu.make_async_copy(aj_hbm.at[r_v.at[par, c]],
                                  j_v.at[par, c], sem_qj).wait()

    def s2(par):
        s1_drain(par)
        for c in range(NCH):
            for k16 in range(128 // LANES):
                sl = pl.ds(k16 * LANES, LANES)
                cidx_v[par, c, sl] = jnp.where(u_v[par, c, sl] < q_v[par, c, sl],
                                               r_v[par, c, sl], j_v[par, c, sl])
        for c in range(NCH):
            pltpu.async_copy(wv_hbm.at[cidx_v.at[par, c]],
                             rows_bufs[par].at[pl.ds(c * 128, 128)], sem_rows)

    def s2_drain(par):
        for c in range(NCH):
            pltpu.make_async_copy(wv_hbm.at[cidx_v.at[par, c]],
                                  rows_bufs[par].at[pl.ds(c * 128, 128)],
                                  sem_rows).wait()

    def s3(k, par, t):
        s2_drain(par)
        rows_ref = rows_bufs[par]
        accs = tuple(jnp.zeros((LANES,), jnp.float32) for _ in range(GROUPS))
        ii = jnp.full((LANES,), k, dtype=jnp.int32)

        def e_body(e, carry):
            accs, w = carry
            mult = plsc.load_gather(ctx_v, [ii, w])
            out = []
            for g in range(GROUPS):
                vals = plsc.load_gather(rows_ref, [lane + g * LANES, w])
                out.append(accs[g] + vals * mult)
            return tuple(out), jnp.bitwise_and(w + 1, EMBED - 1)

        accs, _ = lax.fori_loop(0, EMBED, e_body, (accs, lane))
        for g in range(GROUPS):
            dots4_v[t, pl.ds(g * LANES, LANES)] = accs[g]

    s1(jnp.int32(0), 0)
    s2(0)
    s1(jnp.int32(1), 1)
    s2(1)
    s1(jnp.int32(2), 2)

    def body(j, carry):
        base = j * 4
        for t in range(4):
            b = base + t
            s2((t + 2) % 4)
            s1(b + 3, (t + 3) % 4)
            s3(b, t % 4, t)
        pltpu.sync_copy(dots4_v, dots_hbm.at[pl.ds(b0 + base, 4)])
        return carry

    lax.fori_loop(0, BPW // 4, body, 0)
    s2_drain(0)
    s2_drain(1)
    s1_drain(2)


_sc_call = pl.kernel(
    _sc_body,
    out_type=[
        jax.ShapeDtypeStruct((BATCH, RPB), jnp.float32),
        jax.ShapeDtypeStruct((BATCH, TPAD), jnp.float32),
        jax.ShapeDtypeStruct((BATCH,), jnp.float32),
    ],
    mesh=plsc.VectorSubcoreMesh(core_axis_name="c", subcore_axis_name="s",
                                num_cores=NC, num_subcores=NS),
    compiler_params=pltpu.CompilerParams(
        use_tc_tiling_on_sc=False, needs_layout_passes=False),
    scratch_types=[
        pltpu.VMEM((BPW,), jnp.int32),
        pltpu.VMEM((BPW,), jnp.int32),
        pltpu.VMEM((BPW, TPAD), jnp.float32),
        pltpu.VMEM((BPW,), jnp.float32),
        pltpu.VMEM((N_TOPICS, EMBED), jnp.float32),
        pltpu.VMEM((BPW, EMBED), jnp.float32),
        pltpu.VMEM((4, NCH, 128), jnp.int32),
        pltpu.VMEM((4, NCH, 128), jnp.float32),
        pltpu.VMEM((4, NCH, 128), jnp.float32),
        pltpu.VMEM((4, NCH, 128), jnp.int32),
        pltpu.VMEM((4, NCH, 128), jnp.int32),
        pltpu.VMEM((RPAD, EMBED), jnp.float32),
        pltpu.VMEM((RPAD, EMBED), jnp.float32),
        pltpu.VMEM((RPAD, EMBED), jnp.float32),
        pltpu.VMEM((RPAD, EMBED), jnp.float32),
        pltpu.VMEM((4, RPB), jnp.float32),
        pltpu.SemaphoreType.DMA,
        pltpu.SemaphoreType.DMA,
        pltpu.SemaphoreType.DMA,
    ],
)


def _tc_body(dots_ref, dw_ref, w_ref, neg_ref, dir_ref):
    w = w_ref[...]
    wn = w * (jnp.float32(BATCH) / jnp.sum(w))
    dots = dots_ref[...]
    t = dots[:, :WINDOW]
    nz = dots[:, WINDOW:]
    log_t = jnp.log(jnp.maximum(1.0 / (1.0 + jnp.exp(-t)), EPSILON))
    log_n = jnp.log(jnp.maximum(1.0 / (1.0 + jnp.exp(nz)), EPSILON))
    neg_row = (jnp.sum(log_t, axis=1, keepdims=True)
               + jnp.sum(log_n, axis=1, keepdims=True))
    neg_ref[...] = jnp.reshape(-jnp.sum(wn * neg_row) / jnp.float32(BATCH), (1, 1))
    dw = dw_ref[...][:, :N_TOPICS]
    m = jnp.max(dw, axis=1, keepdims=True)
    lse = m + jnp.log(jnp.sum(jnp.exp(dw - m), axis=1, keepdims=True))
    row = jnp.sum(dw, axis=1, keepdims=True) - jnp.float32(N_TOPICS) * lse
    dir_ref[...] = jnp.reshape(jnp.sum(wn * row) / jnp.float32(BATCH)
                               * jnp.float32(LAMBDA_CONST * (1.0 - 1.0 / N_TOPICS)),
                               (1, 1))


_tc_call = pl.pallas_call(
    _tc_body,
    out_shape=[jax.ShapeDtypeStruct((1, 1), jnp.float32),
               jax.ShapeDtypeStruct((1, 1), jnp.float32)],
)


def kernel(doc_indices, pivot_words, target_words, word_vectors,
           doc_weights_table, topic_vectors, loss_weights, alias_q, alias_J):
    n = BATCH * WINDOW * NUM_SAMPLED
    key = jax.random.key(12345)
    k1, k2 = jax.random.split(key)
    r = jax.random.randint(k1, (n,), 0, VOCAB).astype(jnp.int32)
    u = jax.random.uniform(k2, (n,), dtype=jnp.float32)
    r3 = r.reshape(BATCH, WINDOW * NUM_SAMPLED)
    u3 = u.reshape(BATCH, WINDOW * NUM_SAMPLED)
    pad_i = jnp.zeros((BATCH, RPAD - RPB), jnp.int32)
    force = jnp.full((BATCH, WINDOW), -1.0, jnp.float32)
    pad_f = jnp.full((BATCH, RPAD - RPB), -1.0, jnp.float32)
    r_comb = jnp.concatenate([target_words.astype(jnp.int32), r3, pad_i],
                             axis=1).reshape(BATCH, NCH, 128)
    u_comb = jnp.concatenate([force, u3, pad_f],
                             axis=1).reshape(BATCH, NCH, 128)
    dwt_pad = jnp.pad(doc_weights_table, ((0, 0), (0, TPAD - N_TOPICS)))

    dots, dwg, wg = _sc_call(
        r_comb, u_comb, doc_indices.astype(jnp.int32),
        pivot_words.astype(jnp.int32), word_vectors, dwt_pad, topic_vectors,
        loss_weights, alias_q, alias_J.astype(jnp.int32))

    neg, dirich = _tc_call(dots, dwg, wg.reshape(BATCH, 1))
    return (neg.reshape(()), dirich.reshape(()))


# speedup vs baseline: 1.0661x; 1.0661x over previous
import functools

import jax
import jax.numpy as jnp
from jax import lax
from jax.experimental import pallas as pl
from jax.experimental.pallas import tpu as pltpu
from jax.experimental.pallas import tpu_sc as plsc

VOCAB = 100000
EMBED = 64
N_TOPICS = 25
NUM_SAMPLED = 15
WINDOW = 20
BATCH = 4096
LAMBDA_CONST = 100.0
EPSILON = 1e-09

NC, NS, LANES = 2, 16, 16
NW = NC * NS
BPW = BATCH // NW
RPB = WINDOW * (1 + NUM_SAMPLED)
RPAD = 384
NCH = RPAD // 128
GROUPS = RPB // LANES
TPAD = 32


_GDN = lax.GatherDimensionNumbers(
    offset_dims=(), collapsed_slice_dims=(0,), start_index_map=(0,))


def _permute(v, idx):
    return lax.gather(v, idx[:, None], _GDN, (1,),
                      mode=lax.GatherScatterMode.PROMISE_IN_BOUNDS)


def _bcast(v, lane):
    return _permute(v, jnp.full((LANES,), lane, dtype=jnp.int32))


def _vsum16(v, lane):
    for sh in (8, 4, 2, 1):
        v = v + _permute(v, lane ^ sh)
    return v


def _vmax16(v, lane):
    for sh in (8, 4, 2, 1):
        v = jnp.maximum(v, _permute(v, lane ^ sh))
    return v


def _sc_body(r_hbm, u_hbm, didx_hbm, pidx_hbm, wv_hbm, dwt_hbm, tv_hbm,
             lw_hbm, aq_hbm, aj_hbm,
             dots_hbm, dwo_hbm, wo_hbm,
             didx_v, pidx_v, dw_v, w_v, tv_v, ctx_v,
             r_v, u_v, q_v, j_v, cidx_v,
             rows0_v, rows1_v, rows2_v, rows3_v, dots4_v,
             sem, sem_qj, sem_rows):
    wid = lax.axis_index("s") * NC + lax.axis_index("c")
    b0 = wid * BPW
    lane = lax.broadcasted_iota(jnp.int32, (LANES,), 0)

    pltpu.sync_copy(didx_hbm.at[pl.ds(b0, BPW)], didx_v)
    pltpu.sync_copy(pidx_hbm.at[pl.ds(b0, BPW)], pidx_v)
    pltpu.sync_copy(tv_hbm, tv_v)
    cp1 = pltpu.async_copy(dwt_hbm.at[didx_v], dw_v, sem)
    cp2 = pltpu.async_copy(lw_hbm.at[didx_v], w_v, sem)
    cp3 = pltpu.async_copy(wv_hbm.at[pidx_v], ctx_v, sem)
    cp1.wait()
    cp2.wait()
    cp3.wait()
    pltpu.sync_copy(dw_v, dwo_hbm.at[pl.ds(b0, BPW)])
    pltpu.sync_copy(w_v, wo_hbm.at[pl.ds(b0, BPW)])

    def ctx_body(i, carry):
        c0 = dw_v[i, pl.ds(0, LANES)]
        c1 = dw_v[i, pl.ds(LANES, LANES)]
        valid1 = lane < (N_TOPICS - LANES)
        neg_big = jnp.float32(-1e30)
        m = _vmax16(jnp.maximum(c0, jnp.where(valid1, c1, neg_big)), lane)
        e0 = jnp.exp(c0 - m)
        e1 = jnp.where(valid1, jnp.exp(c1 - m), jnp.float32(0.0))
        inv = 1.0 / _vsum16(e0 + e1, lane)
        mix = [jnp.zeros((LANES,), jnp.float32) for _ in range(EMBED // LANES)]
        for t in range(N_TOPICS):
            src = e0 if t < LANES else e1
            ln = t if t < LANES else t - LANES
            p = _bcast(src, ln) * inv
            for j in range(EMBED // LANES):
                mix[j] = mix[j] + p * tv_v[t, pl.ds(LANES * j, LANES)]
        for j in range(EMBED // LANES):
            sl = pl.ds(LANES * j, LANES)
            ctx_v[i, sl] = mix[j] + ctx_v[i, sl]
        return carry

    lax.fori_loop(0, BPW, ctx_body, 0)

    rows_bufs = (rows0_v, rows1_v, rows2_v, rows3_v)

    def s1(k, par):
        kk = jnp.minimum(k, BPW - 1)
        pltpu.sync_copy(r_hbm.at[b0 + kk], r_v.at[par])
        pltpu.sync_copy(u_hbm.at[b0 + kk], u_v.at[par])
        pltpu.async_copy(aq_hbm.at[r_v.at[par]], q_v.at[par], sem_qj)
        pltpu.async_copy(aj_hbm.at[r_v.at[par]], j_v.at[par], sem_qj)

    def s1_drain(par):
        pltpu.make_async_copy(aq_hbm.at[r_v.at[par]],
                              q_v.at[par], sem_qj).wait()
        pltpu.make_async_copy(aj_hbm.at[r_v.at[par]],
                              j_v.at[par], sem_qj).wait()

    def s2(par):
        s1_drain(par)
        for k16 in range(RPAD // LANES):
            sl = pl.ds(k16 * LANES, LANES)
            cidx_v[par, sl] = jnp.where(u_v[par, sl] < q_v[par, sl],
                                        r_v[par, sl], j_v[par, sl])
        pltpu.async_copy(wv_hbm.at[cidx_v.at[par]], rows_bufs[par], sem_rows)

    def s2_drain(par):
        pltpu.make_async_copy(wv_hbm.at[cidx_v.at[par]], rows_bufs[par],
                              sem_rows).wait()

    def s3(k, par, t):
        s2_drain(par)
        rows_ref = rows_bufs[par]
        accs = tuple(jnp.zeros((LANES,), jnp.float32) for _ in range(GROUPS))
        ii = jnp.full((LANES,), k, dtype=jnp.int32)

        def e_body(e, carry):
            accs, w = carry
            mult = plsc.load_gather(ctx_v, [ii, w])
            out = []
            for g in range(GROUPS):
                vals = plsc.load_gather(rows_ref, [lane + g * LANES, w])
                out.append(accs[g] + vals * mult)
            return tuple(out), jnp.bitwise_and(w + 1, EMBED - 1)

        accs, _ = lax.fori_loop(0, EMBED, e_body, (accs, lane))
        for g in range(GROUPS):
            dots4_v[t, pl.ds(g * LANES, LANES)] = accs[g]

    s1(jnp.int32(0), 0)
    s2(0)
    s1(jnp.int32(1), 1)
    s2(1)
    s1(jnp.int32(2), 2)

    def body(j, carry):
        base = j * 4
        for t in range(4):
            b = base + t
            s2((t + 2) % 4)
            s1(b + 3, (t + 3) % 4)
            s3(b, t % 4, t)
        pltpu.sync_copy(dots4_v, dots_hbm.at[pl.ds(b0 + base, 4)])
        return carry

    lax.fori_loop(0, BPW // 4, body, 0)
    s2_drain(0)
    s2_drain(1)
    s1_drain(2)


_sc_call = pl.kernel(
    _sc_body,
    out_type=[
        jax.ShapeDtypeStruct((BATCH, RPB), jnp.float32),
        jax.ShapeDtypeStruct((BATCH, TPAD), jnp.float32),
        jax.ShapeDtypeStruct((BATCH,), jnp.float32),
    ],
    mesh=plsc.VectorSubcoreMesh(core_axis_name="c", subcore_axis_name="s",
                                num_cores=NC, num_subcores=NS),
    compiler_params=pltpu.CompilerParams(
        use_tc_tiling_on_sc=False, needs_layout_passes=False),
    scratch_types=[
        pltpu.VMEM((BPW,), jnp.int32),
        pltpu.VMEM((BPW,), jnp.int32),
        pltpu.VMEM((BPW, TPAD), jnp.float32),
        pltpu.VMEM((BPW,), jnp.float32),
        pltpu.VMEM((N_TOPICS, EMBED), jnp.float32),
        pltpu.VMEM((BPW, EMBED), jnp.float32),
        pltpu.VMEM((4, RPAD), jnp.int32),
        pltpu.VMEM((4, RPAD), jnp.float32),
        pltpu.VMEM((4, RPAD), jnp.float32),
        pltpu.VMEM((4, RPAD), jnp.int32),
        pltpu.VMEM((4, RPAD), jnp.int32),
        pltpu.VMEM((RPAD, EMBED), jnp.float32),
        pltpu.VMEM((RPAD, EMBED), jnp.float32),
        pltpu.VMEM((RPAD, EMBED), jnp.float32),
        pltpu.VMEM((RPAD, EMBED), jnp.float32),
        pltpu.VMEM((4, RPB), jnp.float32),
        pltpu.SemaphoreType.DMA,
        pltpu.SemaphoreType.DMA,
        pltpu.SemaphoreType.DMA,
    ],
)


def _tc_body(dots_ref, dw_ref, w_ref, neg_ref, dir_ref):
    w = w_ref[...]
    wn = w * (jnp.float32(BATCH) / jnp.sum(w))
    dots = dots_ref[...]
    t = dots[:, :WINDOW]
    nz = dots[:, WINDOW:]
    log_t = jnp.log(jnp.maximum(1.0 / (1.0 + jnp.exp(-t)), EPSILON))
    log_n = jnp.log(jnp.maximum(1.0 / (1.0 + jnp.exp(nz)), EPSILON))
    neg_row = (jnp.sum(log_t, axis=1, keepdims=True)
               + jnp.sum(log_n, axis=1, keepdims=True))
    neg_ref[...] = jnp.reshape(-jnp.sum(wn * neg_row) / jnp.float32(BATCH), (1, 1))
    dw = dw_ref[...][:, :N_TOPICS]
    m = jnp.max(dw, axis=1, keepdims=True)
    lse = m + jnp.log(jnp.sum(jnp.exp(dw - m), axis=1, keepdims=True))
    row = jnp.sum(dw, axis=1, keepdims=True) - jnp.float32(N_TOPICS) * lse
    dir_ref[...] = jnp.reshape(jnp.sum(wn * row) / jnp.float32(BATCH)
                               * jnp.float32(LAMBDA_CONST * (1.0 - 1.0 / N_TOPICS)),
                               (1, 1))


_tc_call = pl.pallas_call(
    _tc_body,
    out_shape=[jax.ShapeDtypeStruct((1, 1), jnp.float32),
               jax.ShapeDtypeStruct((1, 1), jnp.float32)],
)


def kernel(doc_indices, pivot_words, target_words, word_vectors,
           doc_weights_table, topic_vectors, loss_weights, alias_q, alias_J):
    n = BATCH * WINDOW * NUM_SAMPLED
    key = jax.random.key(12345)
    k1, k2 = jax.random.split(key)
    r = jax.random.randint(k1, (n,), 0, VOCAB).astype(jnp.int32)
    u = jax.random.uniform(k2, (n,), dtype=jnp.float32)
    r3 = r.reshape(BATCH, WINDOW * NUM_SAMPLED)
    u3 = u.reshape(BATCH, WINDOW * NUM_SAMPLED)
    pad_i = jnp.zeros((BATCH, RPAD - RPB), jnp.int32)
    force = jnp.full((BATCH, WINDOW), -1.0, jnp.float32)
    pad_f = jnp.full((BATCH, RPAD - RPB), -1.0, jnp.float32)
    r_comb = jnp.concatenate([target_words.astype(jnp.int32), r3, pad_i],
                             axis=1)
    u_comb = jnp.concatenate([force, u3, pad_f], axis=1)
    dwt_pad = jnp.pad(doc_weights_table, ((0, 0), (0, TPAD - N_TOPICS)))

    dots, dwg, wg = _sc_call(
        r_comb, u_comb, doc_indices.astype(jnp.int32),
        pivot_words.astype(jnp.int32), word_vectors, dwt_pad, topic_vectors,
        loss_weights, alias_q, alias_J.astype(jnp.int32))

    neg, dirich = _tc_call(dots, dwg, wg.reshape(BATCH, 1))
    return (neg.reshape(()), dirich.reshape(()))


# speedup vs baseline: 8.6923x; 8.1536x over previous
import functools

import jax
import jax.numpy as jnp
from jax import lax
from jax.experimental import pallas as pl
from jax.experimental.pallas import tpu as pltpu
from jax.experimental.pallas import tpu_sc as plsc

VOCAB = 100000
EMBED = 64
N_TOPICS = 25
NUM_SAMPLED = 15
WINDOW = 20
BATCH = 4096
LAMBDA_CONST = 100.0
EPSILON = 1e-09

NC, NS, LANES = 2, 16, 16
NW = NC * NS
BPW = BATCH // NW
RPB = WINDOW * (1 + NUM_SAMPLED)
RPAD = RPB
GROUPS = RPB // LANES
TPAD = 32


_GDN = lax.GatherDimensionNumbers(
    offset_dims=(), collapsed_slice_dims=(0,), start_index_map=(0,))


def _permute(v, idx):
    return lax.gather(v, idx[:, None], _GDN, (1,),
                      mode=lax.GatherScatterMode.PROMISE_IN_BOUNDS)


def _bcast(v, lane):
    return _permute(v, jnp.full((LANES,), lane, dtype=jnp.int32))


def _vsum16(v, lane):
    for sh in (8, 4, 2, 1):
        v = v + _permute(v, lane ^ sh)
    return v


def _vmax16(v, lane):
    for sh in (8, 4, 2, 1):
        v = jnp.maximum(v, _permute(v, lane ^ sh))
    return v


def _sc_body(r_hbm, u_hbm, didx_hbm, pidx_hbm, wv_hbm, dwt_hbm, tv_hbm,
             lw_hbm, aq_hbm, aj_hbm,
             dots_hbm, dwo_hbm, wo_hbm,
             didx_v, pidx_v, dw_v, w_v, tv_v, ctx_v,
             r_v, u_v, q_v, j_v, cidx_v,
             rows0_v, rows1_v, rows2_v, rows3_v, dots4_v,
             sem, sem_qj, sem_rows):
    wid = lax.axis_index("s") * NC + lax.axis_index("c")
    b0 = wid * BPW
    lane = lax.broadcasted_iota(jnp.int32, (LANES,), 0)

    pltpu.sync_copy(didx_hbm.at[pl.ds(b0, BPW)], didx_v)
    pltpu.sync_copy(pidx_hbm.at[pl.ds(b0, BPW)], pidx_v)
    pltpu.sync_copy(tv_hbm, tv_v)
    cp1 = pltpu.async_copy(dwt_hbm.at[didx_v], dw_v, sem)
    cp2 = pltpu.async_copy(lw_hbm.at[didx_v], w_v, sem)
    cp3 = pltpu.async_copy(wv_hbm.at[pidx_v], ctx_v, sem)
    cp1.wait()
    cp2.wait()
    cp3.wait()
    pltpu.sync_copy(dw_v, dwo_hbm.at[pl.ds(b0, BPW)])
    pltpu.sync_copy(w_v, wo_hbm.at[pl.ds(b0, BPW)])

    def ctx_body(i, carry):
        c0 = dw_v[i, pl.ds(0, LANES)]
        c1 = dw_v[i, pl.ds(LANES, LANES)]
        valid1 = lane < (N_TOPICS - LANES)
        neg_big = jnp.float32(-1e30)
        m = _vmax16(jnp.maximum(c0, jnp.where(valid1, c1, neg_big)), lane)
        e0 = jnp.exp(c0 - m)
        e1 = jnp.where(valid1, jnp.exp(c1 - m), jnp.float32(0.0))
        inv = 1.0 / _vsum16(e0 + e1, lane)
        mix = [jnp.zeros((LANES,), jnp.float32) for _ in range(EMBED // LANES)]
        for t in range(N_TOPICS):
            src = e0 if t < LANES else e1
            ln = t if t < LANES else t - LANES
            p = _bcast(src, ln) * inv
            for j in range(EMBED // LANES):
                mix[j] = mix[j] + p * tv_v[t, pl.ds(LANES * j, LANES)]
        for j in range(EMBED // LANES):
            sl = pl.ds(LANES * j, LANES)
            ctx_v[i, sl] = mix[j] + ctx_v[i, sl]
        return carry

    lax.fori_loop(0, BPW, ctx_body, 0)

    rows_bufs = (rows0_v, rows1_v, rows2_v, rows3_v)

    def s1(k, par):
        kk = jnp.minimum(k, BPW - 1)
        pltpu.sync_copy(r_hbm.at[b0 + kk], r_v.at[par])
        pltpu.sync_copy(u_hbm.at[b0 + kk], u_v.at[par])
        pltpu.async_copy(aq_hbm.at[r_v.at[par]], q_v.at[par], sem_qj)
        pltpu.async_copy(aj_hbm.at[r_v.at[par]], j_v.at[par], sem_qj)

    def s1_drain(par):
        pltpu.make_async_copy(aq_hbm.at[r_v.at[par]],
                              q_v.at[par], sem_qj).wait()
        pltpu.make_async_copy(aj_hbm.at[r_v.at[par]],
                              j_v.at[par], sem_qj).wait()

    def s2(par):
        s1_drain(par)
        for k16 in range(RPAD // LANES):
            sl = pl.ds(k16 * LANES, LANES)
            cidx_v[par, sl] = jnp.where(u_v[par, sl] < q_v[par, sl],
                                        r_v[par, sl], j_v[par, sl])
        pltpu.async_copy(wv_hbm.at[cidx_v.at[par]], rows_bufs[par], sem_rows)

    def s2_drain(par):
        pltpu.make_async_copy(wv_hbm.at[cidx_v.at[par]], rows_bufs[par],
                              sem_rows).wait()

    def s3(k, par, t):
        s2_drain(par)
        rows_ref = rows_bufs[par]
        accs = tuple(jnp.zeros((LANES,), jnp.float32) for _ in range(GROUPS))
        ii = jnp.full((LANES,), k, dtype=jnp.int32)

        def e_body(e, carry):
            accs, w = carry
            mult = plsc.load_gather(ctx_v, [ii, w])
            out = []
            for g in range(GROUPS):
                vals = plsc.load_gather(rows_ref, [lane + g * LANES, w])
                out.append(accs[g] + vals * mult)
            return tuple(out), jnp.bitwise_and(w + 1, EMBED - 1)

        accs, _ = lax.fori_loop(0, EMBED, e_body, (accs, lane))
        for g in range(GROUPS):
            dots4_v[t, pl.ds(g * LANES, LANES)] = accs[g]

    s1(jnp.int32(0), 0)
    s2(0)
    s1(jnp.int32(1), 1)
    s2(1)
    s1(jnp.int32(2), 2)

    def body(j, carry):
        base = j * 4
        for t in range(4):
            b = base + t
            s2((t + 2) % 4)
            s1(b + 3, (t + 3) % 4)
            s3(b, t % 4, t)
        pltpu.sync_copy(dots4_v, dots_hbm.at[pl.ds(b0 + base, 4)])
        return carry

    lax.fori_loop(0, BPW // 4, body, 0)
    s2_drain(0)
    s2_drain(1)
    s1_drain(2)


_sc_call = pl.kernel(
    _sc_body,
    out_type=[
        jax.ShapeDtypeStruct((BATCH, RPB), jnp.float32),
        jax.ShapeDtypeStruct((BATCH, TPAD), jnp.float32),
        jax.ShapeDtypeStruct((BATCH,), jnp.float32),
    ],
    mesh=plsc.VectorSubcoreMesh(core_axis_name="c", subcore_axis_name="s",
                                num_cores=NC, num_subcores=NS),
    compiler_params=pltpu.CompilerParams(
        use_tc_tiling_on_sc=False, needs_layout_passes=False),
    scratch_types=[
        pltpu.VMEM((BPW,), jnp.int32),
        pltpu.VMEM((BPW,), jnp.int32),
        pltpu.VMEM((BPW, TPAD), jnp.float32),
        pltpu.VMEM((BPW,), jnp.float32),
        pltpu.VMEM((N_TOPICS, EMBED), jnp.float32),
        pltpu.VMEM((BPW, EMBED), jnp.float32),
        pltpu.VMEM((4, RPAD), jnp.int32),
        pltpu.VMEM((4, RPAD), jnp.float32),
        pltpu.VMEM((4, RPAD), jnp.float32),
        pltpu.VMEM((4, RPAD), jnp.int32),
        pltpu.VMEM((4, RPAD), jnp.int32),
        pltpu.VMEM((RPAD, EMBED), jnp.float32),
        pltpu.VMEM((RPAD, EMBED), jnp.float32),
        pltpu.VMEM((RPAD, EMBED), jnp.float32),
        pltpu.VMEM((RPAD, EMBED), jnp.float32),
        pltpu.VMEM((4, RPB), jnp.float32),
        pltpu.SemaphoreType.DMA,
        pltpu.SemaphoreType.DMA,
        pltpu.SemaphoreType.DMA,
    ],
)


def _tc_body(dots_ref, dw_ref, w_ref, neg_ref, dir_ref):
    w = w_ref[...]
    wn = w * (jnp.float32(BATCH) / jnp.sum(w))
    dots = dots_ref[...]
    t = dots[:, :WINDOW]
    nz = dots[:, WINDOW:]
    log_t = jnp.log(jnp.maximum(1.0 / (1.0 + jnp.exp(-t)), EPSILON))
    log_n = jnp.log(jnp.maximum(1.0 / (1.0 + jnp.exp(nz)), EPSILON))
    neg_row = (jnp.sum(log_t, axis=1, keepdims=True)
               + jnp.sum(log_n, axis=1, keepdims=True))
    neg_ref[...] = jnp.reshape(-jnp.sum(wn * neg_row) / jnp.float32(BATCH), (1, 1))
    dw = dw_ref[...][:, :N_TOPICS]
    m = jnp.max(dw, axis=1, keepdims=True)
    lse = m + jnp.log(jnp.sum(jnp.exp(dw - m), axis=1, keepdims=True))
    row = jnp.sum(dw, axis=1, keepdims=True) - jnp.float32(N_TOPICS) * lse
    dir_ref[...] = jnp.reshape(jnp.sum(wn * row) / jnp.float32(BATCH)
                               * jnp.float32(LAMBDA_CONST * (1.0 - 1.0 / N_TOPICS)),
                               (1, 1))


_tc_call = pl.pallas_call(
    _tc_body,
    out_shape=[jax.ShapeDtypeStruct((1, 1), jnp.float32),
               jax.ShapeDtypeStruct((1, 1), jnp.float32)],
)


def kernel(doc_indices, pivot_words, target_words, word_vectors,
           doc_weights_table, topic_vectors, loss_weights, alias_q, alias_J):
    n = BATCH * WINDOW * NUM_SAMPLED
    key = jax.random.key(12345)
    k1, k2 = jax.random.split(key)
    r = jax.random.randint(k1, (n,), 0, VOCAB).astype(jnp.int32)
    u = jax.random.uniform(k2, (n,), dtype=jnp.float32)
    r3 = r.reshape(BATCH, WINDOW * NUM_SAMPLED)
    u3 = u.reshape(BATCH, WINDOW * NUM_SAMPLED)
    pad_i = jnp.zeros((BATCH, RPAD - RPB), jnp.int32)
    force = jnp.full((BATCH, WINDOW), -1.0, jnp.float32)
    pad_f = jnp.full((BATCH, RPAD - RPB), -1.0, jnp.float32)
    r_comb = jnp.concatenate([target_words.astype(jnp.int32), r3, pad_i],
                             axis=1)
    u_comb = jnp.concatenate([force, u3, pad_f], axis=1)
    dwt_pad = jnp.pad(doc_weights_table, ((0, 0), (0, TPAD - N_TOPICS)))

    dots, dwg, wg = _sc_call(
        r_comb, u_comb, doc_indices.astype(jnp.int32),
        pivot_words.astype(jnp.int32), word_vectors, dwt_pad, topic_vectors,
        loss_weights, alias_q, alias_J.astype(jnp.int32))

    neg, dirich = _tc_call(dots, dwg, wg.reshape(BATCH, 1))
    return (neg.reshape(()), dirich.reshape(()))
